# baseline (speedup 1.0000x reference)
import functools

import jax
import jax.numpy as jnp
from jax import lax
from jax.experimental import pallas as pl
from jax.experimental.pallas import tpu as pltpu
from jax.experimental.pallas import tpu_sc as plsc


def _make_sc_gather(V, D, B):
    info = plsc.get_sparse_core_info()
    NC, NS = info.num_cores, info.num_subcores
    NW = NC * NS
    assert B % (8 * NW) == 0 and D % info.num_lanes == 0
    b_per_w = B // NW
    mesh = plsc.VectorSubcoreMesh(core_axis_name="c", subcore_axis_name="s")

    @functools.partial(
        pl.kernel,
        mesh=mesh,
        out_type=jax.ShapeDtypeStruct((B, D), jnp.float32),
        scratch_types=[
            pltpu.VMEM((b_per_w,), jnp.int32),
            pltpu.VMEM((b_per_w, D), jnp.float32),
            pltpu.SemaphoreType.DMA,
        ],
        compiler_params=pltpu.CompilerParams(use_tc_tiling_on_sc=False),
    )
    def gather(table_hbm, idx_hbm, out_hbm, idx_v, rows_v, sem):
        wid = lax.axis_index("s") * NC + lax.axis_index("c")
        base = wid * b_per_w
        pltpu.sync_copy(idx_hbm.at[pl.ds(base, b_per_w)], idx_v)
        pltpu.async_copy(table_hbm.at[idx_v], rows_v, sem).wait()
        pltpu.sync_copy(rows_v, out_hbm.at[pl.ds(base, b_per_w)])

    return gather


_VBLK = 2048


def _proj_body(e_ref, w_ref, b_ref, o_ref):
    o_ref[...] = (
        lax.dot_general(
            e_ref[...], w_ref[...],
            (((1,), (1,)), ((), ())),
            preferred_element_type=jnp.float32,
        )
        + b_ref[...]
    )


def _make_tc_proj(B, D, V):
    grid = (pl.cdiv(V, _VBLK),)
    return pl.pallas_call(
        _proj_body,
        grid=grid,
        in_specs=[
            pl.BlockSpec((B, D), lambda j: (0, 0)),
            pl.BlockSpec((_VBLK, D), lambda j: (j, 0)),
            pl.BlockSpec((1, _VBLK), lambda j: (0, j)),
        ],
        out_specs=pl.BlockSpec((B, _VBLK), lambda j: (0, j)),
        out_shape=jax.ShapeDtypeStruct((B, V), jnp.float32),
        compiler_params=pltpu.CompilerParams(
            dimension_semantics=("arbitrary",),
        ),
    )


def kernel(x, embed_table, fc_w, fc_b):
    V, D = embed_table.shape
    (B,) = x.shape
    e = _make_sc_gather(V, D, B)(embed_table, x)
    b2 = fc_b.reshape(1, V)
    return _make_tc_proj(B, D, V)(e, fc_w, b2)


# speedup vs baseline: 2.8207x; 2.8207x over previous
import functools

import jax
import jax.numpy as jnp
from jax import lax
from jax.experimental import pallas as pl
from jax.experimental.pallas import tpu as pltpu
from jax.experimental.pallas import tpu_sc as plsc


def _make_sc_gather(V, D, B):
    info = plsc.get_sparse_core_info()
    NC, NS, L = info.num_cores, info.num_subcores, info.num_lanes
    NW = NC * NS
    assert B % (8 * NW) == 0 and D % L == 0 and V % 2 == 0
    b_per_w = B // NW
    mesh = plsc.VectorSubcoreMesh(core_axis_name="c", subcore_axis_name="s")

    @functools.partial(
        pl.kernel,
        mesh=mesh,
        out_type=jax.ShapeDtypeStruct((B, 2 * D), jnp.float32),
        scratch_types=[
            pltpu.VMEM((b_per_w,), jnp.int32),
            pltpu.VMEM((b_per_w,), jnp.int32),
            pltpu.VMEM((b_per_w, 2 * D), jnp.float32),
            pltpu.SemaphoreType.DMA,
        ],
        compiler_params=pltpu.CompilerParams(needs_layout_passes=False),
    )
    def gather(table2_hbm, idx_hbm, out_hbm, idx_v, idxh_v, rows_v, sem):
        wid = lax.axis_index("s") * NC + lax.axis_index("c")
        base = wid * b_per_w
        pltpu.sync_copy(idx_hbm.at[pl.ds(base, b_per_w)], idx_v)
        for c in range(b_per_w // L):
            v = idx_v[pl.ds(c * L, L)]
            idxh_v[pl.ds(c * L, L)] = lax.shift_right_logical(v, 1)
        pltpu.async_copy(table2_hbm.at[idxh_v], rows_v, sem).wait()
        pltpu.sync_copy(rows_v, out_hbm.at[pl.ds(base, b_per_w)])

    return gather


_VBLK = 2048


def _proj_body(wt_ref, ep_ref, par_ref, b_ref, o_ref):
    D = wt_ref.shape[0]
    w_aug = jnp.concatenate([wt_ref[...], b_ref[...]], axis=0)
    ep = ep_ref[...]
    e = jnp.where(par_ref[...] > 0, ep[:, D:], ep[:, :D])
    e_aug = jnp.concatenate(
        [e, jnp.ones((e.shape[0], 1), jnp.float32)], axis=1
    )
    o_ref[...] = lax.dot_general(
        w_aug, e_aug,
        (((0,), (1,)), ((), ())),
        preferred_element_type=jnp.float32,
    )


def _make_tc_proj(B, D, V):
    grid = (pl.cdiv(V, _VBLK),)
    return pl.pallas_call(
        _proj_body,
        grid=grid,
        in_specs=[
            pl.BlockSpec((D, _VBLK), lambda j: (0, j)),
            pl.BlockSpec((B, 2 * D), lambda j: (0, 0)),
            pl.BlockSpec((B, 1), lambda j: (0, 0)),
            pl.BlockSpec((1, _VBLK), lambda j: (0, j)),
        ],
        out_specs=pl.BlockSpec((_VBLK, B), lambda j: (j, 0)),
        out_shape=jax.ShapeDtypeStruct((V, B), jnp.float32),
        compiler_params=pltpu.CompilerParams(
            dimension_semantics=("arbitrary",),
        ),
    )


def kernel(x, embed_table, fc_w, fc_b):
    V, D = embed_table.shape
    (B,) = x.shape
    table2 = embed_table.reshape(V // 2, 2 * D)
    e2p = _make_sc_gather(V, D, B)(table2, x)
    parity = (x & 1).reshape(B, 1)
    out_t = _make_tc_proj(B, D, V)(fc_w.T, e2p, parity, fc_b.reshape(1, V))
    return out_t.T


# speedup vs baseline: 3.0023x; 1.0644x over previous
import functools

import jax
import jax.numpy as jnp
from jax import lax
from jax.experimental import pallas as pl
from jax.experimental.pallas import tpu as pltpu
from jax.experimental.pallas import tpu_sc as plsc


_TBLK = 2048


def _widen_body(tt_ref, o_ref):
    at = jnp.transpose(tt_ref[...], (1, 0))
    o_ref[...] = jnp.concatenate([at, at], axis=1)


def _make_tc_widen(D, V):
    grid = (pl.cdiv(V, _TBLK),)
    return pl.pallas_call(
        _widen_body,
        grid=grid,
        in_specs=[pl.BlockSpec((D, _TBLK), lambda j: (0, j))],
        out_specs=pl.BlockSpec((_TBLK, 2 * D), lambda j: (j, 0)),
        out_shape=jax.ShapeDtypeStruct((V, 2 * D), jnp.float32),
        compiler_params=pltpu.CompilerParams(
            dimension_semantics=("arbitrary",),
        ),
    )


def _make_sc_gather(V, D, B):
    info = plsc.get_sparse_core_info()
    NC, NS, L = info.num_cores, info.num_subcores, info.num_lanes
    NW = NC * NS
    assert B % (8 * NW) == 0 and (2 * D) % (8 * L) == 0
    b_per_w = B // NW
    mesh = plsc.VectorSubcoreMesh(core_axis_name="c", subcore_axis_name="s")

    @functools.partial(
        pl.kernel,
        mesh=mesh,
        out_type=jax.ShapeDtypeStruct((B, 2 * D), jnp.float32),
        scratch_types=[
            pltpu.VMEM((b_per_w,), jnp.int32),
            pltpu.VMEM((b_per_w, 2 * D), jnp.float32),
            pltpu.SemaphoreType.DMA,
        ],
        compiler_params=pltpu.CompilerParams(needs_layout_passes=False),
    )
    def gather(tablew_hbm, idx_hbm, out_hbm, idx_v, rows_v, sem):
        wid = lax.axis_index("s") * NC + lax.axis_index("c")
        base = wid * b_per_w
        pltpu.sync_copy(idx_hbm.at[pl.ds(base, b_per_w)], idx_v)
        pltpu.async_copy(tablew_hbm.at[idx_v], rows_v, sem).wait()
        pltpu.sync_copy(rows_v, out_hbm.at[pl.ds(base, b_per_w)])

    return gather


_VBLK = 2048


def _proj_body(wt_ref, ep_ref, b_ref, o_ref):
    D = wt_ref.shape[0]
    w_aug = jnp.concatenate([wt_ref[...], b_ref[...]], axis=0)
    e = ep_ref[...][:, :D]
    e_aug = jnp.concatenate(
        [e, jnp.ones((e.shape[0], 1), jnp.float32)], axis=1
    )
    o_ref[...] = lax.dot_general(
        w_aug, e_aug,
        (((0,), (1,)), ((), ())),
        preferred_element_type=jnp.float32,
    )


def _make_tc_proj(B, D, V):
    grid = (pl.cdiv(V, _VBLK),)
    return pl.pallas_call(
        _proj_body,
        grid=grid,
        in_specs=[
            pl.BlockSpec((D, _VBLK), lambda j: (0, j)),
            pl.BlockSpec((B, 2 * D), lambda j: (0, 0)),
            pl.BlockSpec((1, _VBLK), lambda j: (0, j)),
        ],
        out_specs=pl.BlockSpec((_VBLK, B), lambda j: (j, 0)),
        out_shape=jax.ShapeDtypeStruct((V, B), jnp.float32),
        compiler_params=pltpu.CompilerParams(
            dimension_semantics=("arbitrary",),
        ),
    )


def kernel(x, embed_table, fc_w, fc_b):
    V, D = embed_table.shape
    (B,) = x.shape
    tablew = _make_tc_widen(D, V)(embed_table.T)
    e2p = _make_sc_gather(V, D, B)(tablew, x)
    out_t = _make_tc_proj(B, D, V)(fc_w.T, e2p, fc_b.reshape(1, V))
    return out_t.T


# speedup vs baseline: 3.0666x; 1.0214x over previous
import functools

import jax
import jax.numpy as jnp
from jax import lax
from jax.experimental import pallas as pl
from jax.experimental.pallas import tpu as pltpu
from jax.experimental.pallas import tpu_sc as plsc


_TBLK = 2048


def _widen_body(tt_ref, eye2_ref, o_ref):
    o_ref[...] = lax.dot_general(
        tt_ref[...], eye2_ref[...],
        (((0,), (0,)), ((), ())),
        preferred_element_type=jnp.float32,
    )


def _make_tc_widen(D, V):
    grid = (pl.cdiv(V, _TBLK),)
    return pl.pallas_call(
        _widen_body,
        grid=grid,
        in_specs=[
            pl.BlockSpec((D, _TBLK), lambda j: (0, j)),
            pl.BlockSpec((D, 2 * D), lambda j: (0, 0)),
        ],
        out_specs=pl.BlockSpec((_TBLK, 2 * D), lambda j: (j, 0)),
        out_shape=jax.ShapeDtypeStruct((V, 2 * D), jnp.float32),
        compiler_params=pltpu.CompilerParams(
            dimension_semantics=("arbitrary",),
        ),
    )


def _make_sc_gather(V, D, B):
    info = plsc.get_sparse_core_info()
    NC, NS, L = info.num_cores, info.num_subcores, info.num_lanes
    NW = NC * NS
    assert B % (8 * NW) == 0 and (2 * D) % (8 * L) == 0
    b_per_w = B // NW
    mesh = plsc.VectorSubcoreMesh(core_axis_name="c", subcore_axis_name="s")

    @functools.partial(
        pl.kernel,
        mesh=mesh,
        out_type=jax.ShapeDtypeStruct((B, 2 * D), jnp.float32),
        scratch_types=[
            pltpu.VMEM((b_per_w,), jnp.int32),
            pltpu.VMEM((b_per_w, 2 * D), jnp.float32),
            pltpu.SemaphoreType.DMA,
        ],
        compiler_params=pltpu.CompilerParams(needs_layout_passes=False),
    )
    def gather(tablew_hbm, idx_hbm, out_hbm, idx_v, rows_v, sem):
        wid = lax.axis_index("s") * NC + lax.axis_index("c")
        base = wid * b_per_w
        pltpu.sync_copy(idx_hbm.at[pl.ds(base, b_per_w)], idx_v)
        pltpu.async_copy(tablew_hbm.at[idx_v], rows_v, sem).wait()
        pltpu.sync_copy(rows_v, out_hbm.at[pl.ds(base, b_per_w)])

    return gather


_VBLK = 2048


def _proj_body(wt_ref, ep_ref, b_ref, o_ref):
    D = wt_ref.shape[0]
    w_aug = jnp.concatenate([wt_ref[...], b_ref[...]], axis=0)
    e = ep_ref[...][:, :D]
    e_aug = jnp.concatenate(
        [e, jnp.ones((e.shape[0], 1), jnp.float32)], axis=1
    )
    o_ref[...] = lax.dot_general(
        w_aug, e_aug,
        (((0,), (1,)), ((), ())),
        preferred_element_type=jnp.float32,
    )


def _make_tc_proj(B, D, V):
    grid = (pl.cdiv(V, _VBLK),)
    return pl.pallas_call(
        _proj_body,
        grid=grid,
        in_specs=[
            pl.BlockSpec((D, _VBLK), lambda j: (0, j)),
            pl.BlockSpec((B, 2 * D), lambda j: (0, 0)),
            pl.BlockSpec((1, _VBLK), lambda j: (0, j)),
        ],
        out_specs=pl.BlockSpec((_VBLK, B), lambda j: (j, 0)),
        out_shape=jax.ShapeDtypeStruct((V, B), jnp.float32),
        compiler_params=pltpu.CompilerParams(
            dimension_semantics=("arbitrary",),
        ),
    )


def kernel(x, embed_table, fc_w, fc_b):
    V, D = embed_table.shape
    (B,) = x.shape
    eye2 = jnp.tile(jnp.eye(D, dtype=jnp.float32), (1, 2))
    tablew = _make_tc_widen(D, V)(embed_table.T, eye2)
    e2p = _make_sc_gather(V, D, B)(tablew, x)
    out_t = _make_tc_proj(B, D, V)(fc_w.T, e2p, fc_b.reshape(1, V))
    return out_t.T


# speedup vs baseline: 3.3835x; 1.1034x over previous
import functools

import jax
import jax.numpy as jnp
from jax import lax
from jax.experimental import pallas as pl
from jax.experimental.pallas import tpu as pltpu
from jax.experimental.pallas import tpu_sc as plsc


_TBLK = 8192


def _widen_body(tt_ref, eye2_ref, o_ref):
    o_ref[...] = lax.dot_general(
        tt_ref[...], eye2_ref[...],
        (((0,), (0,)), ((), ())),
        preferred_element_type=jnp.float32,
    )


def _make_tc_widen(D, V):
    grid = (pl.cdiv(V, _TBLK),)
    return pl.pallas_call(
        _widen_body,
        grid=grid,
        in_specs=[
            pl.BlockSpec((D, _TBLK), lambda j: (0, j)),
            pl.BlockSpec((D, 2 * D), lambda j: (0, 0)),
        ],
        out_specs=pl.BlockSpec((_TBLK, 2 * D), lambda j: (j, 0)),
        out_shape=jax.ShapeDtypeStruct((V, 2 * D), jnp.float32),
        compiler_params=pltpu.CompilerParams(
            dimension_semantics=("arbitrary",),
        ),
    )


def _make_sc_gather(V, D, B):
    info = plsc.get_sparse_core_info()
    NC, NS, L = info.num_cores, info.num_subcores, info.num_lanes
    NW = NC * NS
    assert B % (8 * NW) == 0 and (2 * D) % (8 * L) == 0
    b_per_w = B // NW
    mesh = plsc.VectorSubcoreMesh(core_axis_name="c", subcore_axis_name="s")

    @functools.partial(
        pl.kernel,
        mesh=mesh,
        out_type=jax.ShapeDtypeStruct((B, 2 * D), jnp.float32),
        scratch_types=[
            pltpu.VMEM((b_per_w,), jnp.int32),
            pltpu.VMEM((b_per_w, 2 * D), jnp.float32),
            pltpu.SemaphoreType.DMA,
        ],
        compiler_params=pltpu.CompilerParams(needs_layout_passes=False),
    )
    def gather(tablew_hbm, idx_hbm, out_hbm, idx_v, rows_v, sem):
        wid = lax.axis_index("s") * NC + lax.axis_index("c")
        base = wid * b_per_w
        pltpu.sync_copy(idx_hbm.at[pl.ds(base, b_per_w)], idx_v)
        pltpu.async_copy(tablew_hbm.at[idx_v], rows_v, sem).wait()
        pltpu.sync_copy(rows_v, out_hbm.at[pl.ds(base, b_per_w)])

    return gather


_VBLK = 2048


def _proj_body(wt_ref, ep_ref, b_ref, o_ref):
    D = wt_ref.shape[0]
    w_aug = jnp.concatenate([wt_ref[...], b_ref[...]], axis=0)
    e = ep_ref[...][:, :D]
    e_aug = jnp.concatenate(
        [e, jnp.ones((e.shape[0], 1), jnp.float32)], axis=1
    )
    o_ref[...] = lax.dot_general(
        w_aug, e_aug,
        (((0,), (1,)), ((), ())),
        preferred_element_type=jnp.float32,
    )


def _make_tc_proj(B, D, V):
    grid = (pl.cdiv(V, _VBLK),)
    return pl.pallas_call(
        _proj_body,
        grid=grid,
        in_specs=[
            pl.BlockSpec((D, _VBLK), lambda j: (0, j)),
            pl.BlockSpec((B, 2 * D), lambda j: (0, 0)),
            pl.BlockSpec((1, _VBLK), lambda j: (0, j)),
        ],
        out_specs=pl.BlockSpec((_VBLK, B), lambda j: (j, 0)),
        out_shape=jax.ShapeDtypeStruct((V, B), jnp.float32),
        compiler_params=pltpu.CompilerParams(
            dimension_semantics=("arbitrary",),
        ),
    )


def kernel(x, embed_table, fc_w, fc_b):
    V, D = embed_table.shape
    (B,) = x.shape
    eye2 = jnp.tile(jnp.eye(D, dtype=jnp.float32), (1, 2))
    tablew = _make_tc_widen(D, V)(embed_table.T, eye2)
    e2p = _make_sc_gather(V, D, B)(tablew, x)
    out_t = _make_tc_proj(B, D, V)(fc_w.T, e2p, fc_b.reshape(1, V))
    return out_t.T


# speedup vs baseline: 3.4017x; 1.0054x over previous
import functools

import jax
import jax.numpy as jnp
from jax import lax
from jax.experimental import pallas as pl
from jax.experimental.pallas import tpu as pltpu
from jax.experimental.pallas import tpu_sc as plsc


_TBLK = 8192


def _widen_body(tt_ref, eye2_ref, o_ref):
    o_ref[...] = lax.dot_general(
        tt_ref[...], eye2_ref[...],
        (((0,), (0,)), ((), ())),
        preferred_element_type=jnp.float32,
    )


def _make_tc_widen(D, V):
    grid = (pl.cdiv(V, _TBLK),)
    return pl.pallas_call(
        _widen_body,
        grid=grid,
        in_specs=[
            pl.BlockSpec((D, _TBLK), lambda j: (0, j)),
            pl.BlockSpec((D, 2 * D), lambda j: (0, 0)),
        ],
        out_specs=pl.BlockSpec((_TBLK, 2 * D), lambda j: (j, 0)),
        out_shape=jax.ShapeDtypeStruct((V, 2 * D), jnp.float32),
        compiler_params=pltpu.CompilerParams(
            dimension_semantics=("parallel",),
        ),
    )


def _make_sc_gather(V, D, B):
    info = plsc.get_sparse_core_info()
    NC, NS, L = info.num_cores, info.num_subcores, info.num_lanes
    NW = NC * NS
    assert B % (8 * NW) == 0 and (2 * D) % (8 * L) == 0
    b_per_w = B // NW
    mesh = plsc.VectorSubcoreMesh(core_axis_name="c", subcore_axis_name="s")

    @functools.partial(
        pl.kernel,
        mesh=mesh,
        out_type=jax.ShapeDtypeStruct((B, 2 * D), jnp.float32),
        scratch_types=[
            pltpu.VMEM((b_per_w,), jnp.int32),
            pltpu.VMEM((b_per_w, 2 * D), jnp.float32),
            pltpu.SemaphoreType.DMA,
        ],
        compiler_params=pltpu.CompilerParams(needs_layout_passes=False),
    )
    def gather(tablew_hbm, idx_hbm, out_hbm, idx_v, rows_v, sem):
        wid = lax.axis_index("s") * NC + lax.axis_index("c")
        base = wid * b_per_w
        pltpu.sync_copy(idx_hbm.at[pl.ds(base, b_per_w)], idx_v)
        pltpu.async_copy(tablew_hbm.at[idx_v], rows_v, sem).wait()
        pltpu.sync_copy(rows_v, out_hbm.at[pl.ds(base, b_per_w)])

    return gather


_VBLK = 2048


def _proj_body(wt_ref, ep_ref, b_ref, o_ref):
    D = wt_ref.shape[0]
    w_aug = jnp.concatenate([wt_ref[...], b_ref[...]], axis=0)
    e = ep_ref[...][:, :D]
    e_aug = jnp.concatenate(
        [e, jnp.ones((e.shape[0], 1), jnp.float32)], axis=1
    )
    o_ref[...] = lax.dot_general(
        w_aug, e_aug,
        (((0,), (1,)), ((), ())),
        preferred_element_type=jnp.float32,
    )


def _make_tc_proj(B, D, V):
    grid = (pl.cdiv(V, _VBLK),)
    return pl.pallas_call(
        _proj_body,
        grid=grid,
        in_specs=[
            pl.BlockSpec((D, _VBLK), lambda j: (0, j)),
            pl.BlockSpec((B, 2 * D), lambda j: (0, 0)),
            pl.BlockSpec((1, _VBLK), lambda j: (0, j)),
        ],
        out_specs=pl.BlockSpec((_VBLK, B), lambda j: (j, 0)),
        out_shape=jax.ShapeDtypeStruct((V, B), jnp.float32),
        compiler_params=pltpu.CompilerParams(
            dimension_semantics=("parallel",),
        ),
    )


def kernel(x, embed_table, fc_w, fc_b):
    V, D = embed_table.shape
    (B,) = x.shape
    eye2 = jnp.tile(jnp.eye(D, dtype=jnp.float32), (1, 2))
    tablew = _make_tc_widen(D, V)(embed_table.T, eye2)
    e2p = _make_sc_gather(V, D, B)(tablew, x)
    out_t = _make_tc_proj(B, D, V)(fc_w.T, e2p, fc_b.reshape(1, V))
    return out_t.T


# speedup vs baseline: 3.4580x; 1.0166x over previous
import functools

import jax
import jax.numpy as jnp
from jax import lax
from jax.experimental import pallas as pl
from jax.experimental.pallas import tpu as pltpu
from jax.experimental.pallas import tpu_sc as plsc


_TBLK = 16384


def _widen_body(tt_ref, eye2_ref, o_ref):
    o_ref[...] = lax.dot_general(
        tt_ref[...], eye2_ref[...],
        (((0,), (0,)), ((), ())),
        preferred_element_type=jnp.float32,
    )


def _make_tc_widen(D, V):
    grid = (pl.cdiv(V, _TBLK),)
    return pl.pallas_call(
        _widen_body,
        grid=grid,
        in_specs=[
            pl.BlockSpec((D, _TBLK), lambda j: (0, j)),
            pl.BlockSpec((D, 2 * D), lambda j: (0, 0)),
        ],
        out_specs=pl.BlockSpec((_TBLK, 2 * D), lambda j: (j, 0)),
        out_shape=jax.ShapeDtypeStruct((V, 2 * D), jnp.float32),
        compiler_params=pltpu.CompilerParams(
            dimension_semantics=("parallel",),
        ),
    )


def _make_sc_gather(V, D, B):
    info = plsc.get_sparse_core_info()
    NC, NS, L = info.num_cores, info.num_subcores, info.num_lanes
    NW = NC * NS
    assert B % (8 * NW) == 0 and (2 * D) % (8 * L) == 0
    b_per_w = B // NW
    mesh = plsc.VectorSubcoreMesh(core_axis_name="c", subcore_axis_name="s")

    @functools.partial(
        pl.kernel,
        mesh=mesh,
        out_type=jax.ShapeDtypeStruct((B, 2 * D), jnp.float32),
        scratch_types=[
            pltpu.VMEM((b_per_w,), jnp.int32),
            pltpu.VMEM((b_per_w, 2 * D), jnp.float32),
            pltpu.SemaphoreType.DMA,
        ],
        compiler_params=pltpu.CompilerParams(needs_layout_passes=False),
    )
    def gather(tablew_hbm, idx_hbm, out_hbm, idx_v, rows_v, sem):
        wid = lax.axis_index("s") * NC + lax.axis_index("c")
        base = wid * b_per_w
        pltpu.sync_copy(idx_hbm.at[pl.ds(base, b_per_w)], idx_v)
        pltpu.async_copy(tablew_hbm.at[idx_v], rows_v, sem).wait()
        pltpu.sync_copy(rows_v, out_hbm.at[pl.ds(base, b_per_w)])

    return gather


_VBLK = 4096


def _proj_body(wt_ref, ep_ref, b_ref, o_ref):
    D = wt_ref.shape[0]
    w_aug = jnp.concatenate([wt_ref[...], b_ref[...]], axis=0)
    e = ep_ref[...][:, :D]
    e_aug = jnp.concatenate(
        [e, jnp.ones((e.shape[0], 1), jnp.float32)], axis=1
    )
    o_ref[...] = lax.dot_general(
        w_aug, e_aug,
        (((0,), (1,)), ((), ())),
        preferred_element_type=jnp.float32,
    )


def _make_tc_proj(B, D, V):
    grid = (pl.cdiv(V, _VBLK),)
    return pl.pallas_call(
        _proj_body,
        grid=grid,
        in_specs=[
            pl.BlockSpec((D, _VBLK), lambda j: (0, j)),
            pl.BlockSpec((B, 2 * D), lambda j: (0, 0)),
            pl.BlockSpec((1, _VBLK), lambda j: (0, j)),
        ],
        out_specs=pl.BlockSpec((_VBLK, B), lambda j: (j, 0)),
        out_shape=jax.ShapeDtypeStruct((V, B), jnp.float32),
        compiler_params=pltpu.CompilerParams(
            dimension_semantics=("parallel",),
        ),
    )


def kernel(x, embed_table, fc_w, fc_b):
    V, D = embed_table.shape
    (B,) = x.shape
    eye2 = jnp.tile(jnp.eye(D, dtype=jnp.float32), (1, 2))
    tablew = _make_tc_widen(D, V)(embed_table.T, eye2)
    e2p = _make_sc_gather(V, D, B)(tablew, x)
    out_t = _make_tc_proj(B, D, V)(fc_w.T, e2p, fc_b.reshape(1, V))
    return out_t.T
